# Initial kernel scaffold; baseline (speedup 1.0000x reference)
import jax
import jax.numpy as jnp
from jax.experimental import pallas as pl


def kernel(x0, x1, edge_index0, edge_index1, W1_0, b1_0, W1_1, b1_1, W2, b2):
    raise NotImplementedError("write your pallas kernel here")


# speedup vs baseline: 7.8486x; 7.8486x over previous
import functools

import jax
import jax.numpy as jnp
from jax import lax
from jax.experimental import pallas as pl
from jax.experimental.pallas import tpu as pltpu
from jax.experimental.pallas import tpu_sc as plsc

N = 10000
E = 160000
F = 256
H = 128
NC = 2
NS = 16
C = 128
NCHUNK = 80
EPAD = NS * NCHUNK * C
NP = 10240
RPT = NP // NS
BM = 1000


@functools.cache
def _mesh():
    return plsc.VectorSubcoreMesh(
        core_axis_name="c", subcore_axis_name="s",
        num_cores=NC, num_subcores=NS)


@functools.cache
def _sc_hist_kernel():
    return pl.kernel(
        _sc_hist_body,
        out_type=(jax.ShapeDtypeStruct((NP, H), jnp.float32),
                  jax.ShapeDtypeStruct((NP, H), jnp.float32)),
        mesh=_mesh(),
        scratch_types=[
            pltpu.VMEM((NCHUNK, C), jnp.int32),
            pltpu.VMEM((C, H), jnp.float32),
            pltpu.VMEM_SHARED((NP, H), jnp.float32),
        ],
    )


def _sc_hist(col0, col1, ones16, zeros16):
    return _sc_hist_kernel()(col0, col1, ones16, zeros16)


def _sc_hist_body(col0_h, col1_h, ones_h, zeros_h, cnt0_h, cnt1_h, colv, onesv, CNT):
    c = lax.axis_index("c")
    s = lax.axis_index("s")
    pltpu.sync_copy(zeros_h, CNT.at[pl.ds(s * RPT, RPT)])
    pltpu.sync_copy(ones_h, onesv)
    plsc.subcore_barrier()
    for cc, colh in ((0, col0_h), (1, col1_h)):
        @pl.when(c == cc)
        def _(colh=colh):
            pltpu.sync_copy(colh.at[s], colv)

            def body(k, carry):
                pltpu.sync_copy(onesv, CNT.at[colv.at[k]], add=True)
                return carry
            lax.fori_loop(0, NCHUNK, body, 0)
    plsc.subcore_barrier()
    for cc, outh in ((0, cnt0_h), (1, cnt1_h)):
        @pl.when(c == cc)
        def _(outh=outh):
            pltpu.sync_copy(CNT.at[pl.ds(s * RPT, RPT)],
                            outh.at[pl.ds(s * RPT, RPT)])


NC2 = NCHUNK // 2


@functools.cache
def _sc_scatter_kernel():
    return pl.kernel(
        _sc_scatter_body,
        out_type=(jax.ShapeDtypeStruct((NP, H), jnp.float32),
                  jax.ShapeDtypeStruct((NP, H), jnp.float32)),
        mesh=_mesh(),
        scratch_types=[
            pltpu.VMEM((NC2, C), jnp.int32),
            pltpu.VMEM((NC2, C), jnp.int32),
            pltpu.VMEM((C, H), jnp.float32),
            pltpu.VMEM((C, H), jnp.float32),
            pltpu.SemaphoreType.DMA,
            pltpu.SemaphoreType.DMA,
            pltpu.VMEM_SHARED((NP, H), jnp.float32),
        ],
    )


def _sc_scatter(row, col, zpa, zpb, zerosH):
    return _sc_scatter_kernel()(row, col, zpa, zpb, zerosH)


def _sc_scatter_body(row_h, col_h, zpa_h, zpb_h, zeros_h, ua_h, ub_h,
                     rowv, colv, buf0, buf1, sem0, sem1, ACC):
    c = lax.axis_index("c")
    s = lax.axis_index("s")
    pltpu.sync_copy(zeros_h, ACC.at[pl.ds(s * RPT, RPT)])
    plsc.subcore_barrier()
    bufs = (buf0, buf1)
    sems = (sem0, sem1)
    for cc, zph in ((0, zpa_h), (1, zpb_h)):
        @pl.when(c == cc)
        def _(zph=zph):
            def start(chunk, b):
                pltpu.async_copy(zph.at[rowv.at[chunk]], bufs[b], sems[b])

            def wait(b):
                pltpu.make_async_copy(zph.at[pl.ds(0, C)], bufs[b],
                                      sems[b]).wait()

            for half in range(2):
                pltpu.sync_copy(row_h.at[s].at[pl.ds(half * NC2, NC2)], rowv)
                pltpu.sync_copy(col_h.at[s].at[pl.ds(half * NC2, NC2)], colv)
                start(0, 0)

                def body(kk, carry):
                    k0 = kk * 2
                    for b in range(2):
                        chunk = k0 + b

                        @pl.when(chunk + 1 < NC2)
                        def _s(chunk=chunk, b=b):
                            start(chunk + 1, 1 - b)
                        wait(b)
                        pltpu.sync_copy(bufs[b], ACC.at[colv.at[chunk]],
                                        add=True)
                    return carry
                lax.fori_loop(0, NC2 // 2, body, 0)
    plsc.subcore_barrier()
    for cc, outh in ((0, ua_h), (1, ub_h)):
        @pl.when(c == cc)
        def _(outh=outh):
            pltpu.sync_copy(ACC.at[pl.ds(s * RPT, RPT)],
                            outh.at[pl.ds(s * RPT, RPT)])


def _dis(cnt_blk):
    return lax.rsqrt(1.0 + cnt_blk[:, :1])


def _mm_scale_body(x_ref, w_ref, cnt_ref, oa_ref, ob_ref):
    xw = jnp.dot(x_ref[...], w_ref[...], preferred_element_type=jnp.float32)
    zp = _dis(cnt_ref[...]) * xw
    oa_ref[...] = zp[:, :H]
    ob_ref[...] = zp[:, H:]


def _mm_scale(x, W, cnt):
    return pl.pallas_call(
        _mm_scale_body,
        grid=(N // BM,),
        in_specs=[
            pl.BlockSpec((BM, F), lambda i: (i, 0)),
            pl.BlockSpec((F, F), lambda i: (0, 0)),
            pl.BlockSpec((BM, H), lambda i: (i, 0)),
        ],
        out_specs=[pl.BlockSpec((BM, H), lambda i: (i, 0))] * 2,
        out_shape=[jax.ShapeDtypeStruct((N, H), jnp.float32)] * 2,
    )(x, W, cnt)


def _mid_body(u0a, u0b, z0a, z0b, u1a, u1b, z1a, z1b, cnt0, cnt1,
              b10, b11, w2, oa_ref, ob_ref):
    dis0 = _dis(cnt0[...])
    dis1 = _dis(cnt1[...])
    h0 = jnp.concatenate([u0a[...] + z0a[...], u0b[...] + z0b[...]], axis=1)
    h0 = jnp.maximum(dis0 * h0 + b10[...], 0.0)
    h1 = jnp.concatenate([u1a[...] + z1a[...], u1b[...] + z1b[...]], axis=1)
    h1 = jnp.maximum(dis1 * h1 + b11[...], 0.0)
    w2 = w2[...]
    t = (jnp.dot(h0, w2[:F], preferred_element_type=jnp.float32)
         + jnp.dot(h1, w2[F:], preferred_element_type=jnp.float32))
    zp2 = dis1 * t
    oa_ref[...] = zp2[:, :H]
    ob_ref[...] = zp2[:, H:]


def _mid(u0a, u0b, z0a, z0b, u1a, u1b, z1a, z1b, cnt0, cnt1, b10, b11, W2):
    blkH = pl.BlockSpec((BM, H), lambda i: (i, 0))
    blkC = pl.BlockSpec((BM, H), lambda i: (i, 0))
    return pl.pallas_call(
        _mid_body,
        grid=(N // BM,),
        in_specs=[blkH] * 8 + [blkC, blkC,
                               pl.BlockSpec((1, F), lambda i: (0, 0)),
                               pl.BlockSpec((1, F), lambda i: (0, 0)),
                               pl.BlockSpec((2 * F, F), lambda i: (0, 0))],
        out_specs=[blkH] * 2,
        out_shape=[jax.ShapeDtypeStruct((N, H), jnp.float32)] * 2,
    )(u0a, u0b, z0a, z0b, u1a, u1b, z1a, z1b, cnt0, cnt1, b10, b11, W2)


def _final_body(u2a, u2b, z2a, z2b, cnt1, b2, o_ref):
    t = jnp.concatenate([u2a[...] + z2a[...], u2b[...] + z2b[...]], axis=1)
    o_ref[...] = _dis(cnt1[...]) * t + b2[...]


def _final(u2a, u2b, z2a, z2b, cnt1, b2):
    blkH = pl.BlockSpec((BM, H), lambda i: (i, 0))
    return pl.pallas_call(
        _final_body,
        grid=(N // BM,),
        in_specs=[blkH] * 4 + [pl.BlockSpec((BM, H), lambda i: (i, 0)),
                               pl.BlockSpec((1, F), lambda i: (0, 0))],
        out_specs=pl.BlockSpec((BM, F), lambda i: (i, 0)),
        out_shape=jax.ShapeDtypeStruct((N, F), jnp.float32),
    )(u2a, u2b, z2a, z2b, cnt1, b2)


def _pad_edges(ei):
    pr = jnp.zeros((EPAD - E,), jnp.int32)
    pc = jnp.full((EPAD - E,), N, jnp.int32)
    row = jnp.concatenate([ei[0], pr]).reshape(NS, NCHUNK, C)
    col = jnp.concatenate([ei[1], pc]).reshape(NS, NCHUNK, C)
    return row, col


def kernel(x0, x1, edge_index0, edge_index1, W1_0, b1_0, W1_1, b1_1, W2, b2):
    row0, col0 = _pad_edges(edge_index0)
    row1, col1 = _pad_edges(edge_index1)
    onesH = jnp.ones((C, H), jnp.float32)
    zerosH = jnp.zeros((RPT, H), jnp.float32)
    b10 = b1_0.reshape(1, F)
    b11 = b1_1.reshape(1, F)
    b2r = b2.reshape(1, F)

    cnt0, cnt1 = _sc_hist(col0, col1, onesH, zerosH)
    cnt0, cnt1 = cnt0[:N], cnt1[:N]
    z0a, z0b = _mm_scale(x0, W1_0, cnt0)
    z1a, z1b = _mm_scale(x1, W1_1, cnt1)
    u0a, u0b = _sc_scatter(row0, col0, z0a, z0b, zerosH)
    u1a, u1b = _sc_scatter(row1, col1, z1a, z1b, zerosH)
    z2a, z2b = _mid(u0a[:N], u0b[:N], z0a, z0b, u1a[:N], u1b[:N], z1a, z1b,
                    cnt0, cnt1, b10, b11, W2)
    u2a, u2b = _sc_scatter(row1, col1, z2a, z2b, zerosH)
    return _final(u2a[:N], u2b[:N], z2a, z2b, cnt1, b2r)


# speedup vs baseline: 7.8495x; 1.0001x over previous
import functools

import jax
import jax.numpy as jnp
from jax import lax
from jax.experimental import pallas as pl
from jax.experimental.pallas import tpu as pltpu
from jax.experimental.pallas import tpu_sc as plsc

N = 10000
E = 160000
F = 256
H = 128
NC = 2
NS = 16
C = 128
NCHUNK = 80
EPAD = NS * NCHUNK * C
NP = 10240
RPT = NP // NS
BM = 1000


@functools.cache
def _mesh():
    return plsc.VectorSubcoreMesh(
        core_axis_name="c", subcore_axis_name="s",
        num_cores=NC, num_subcores=NS)


@functools.cache
def _sc_hist_kernel():
    return pl.kernel(
        _sc_hist_body,
        out_type=(jax.ShapeDtypeStruct((NP, H), jnp.float32),
                  jax.ShapeDtypeStruct((NP, H), jnp.float32)),
        mesh=_mesh(),
        scratch_types=[
            pltpu.VMEM((NCHUNK, C), jnp.int32),
            pltpu.VMEM((C, H), jnp.float32),
            pltpu.VMEM_SHARED((NP, H), jnp.float32),
        ],
    )


def _sc_hist(col0, col1, ones16, zeros16):
    return _sc_hist_kernel()(col0, col1, ones16, zeros16)


def _sc_hist_body(col0_h, col1_h, ones_h, zeros_h, cnt0_h, cnt1_h, colv, onesv, CNT):
    c = lax.axis_index("c")
    s = lax.axis_index("s")
    pltpu.sync_copy(zeros_h, CNT.at[pl.ds(s * RPT, RPT)])
    pltpu.sync_copy(ones_h, onesv)
    plsc.subcore_barrier()
    for cc, colh in ((0, col0_h), (1, col1_h)):
        @pl.when(c == cc)
        def _(colh=colh):
            pltpu.sync_copy(colh.at[s], colv)

            def body(k, carry):
                pltpu.sync_copy(onesv, CNT.at[colv.at[k]], add=True)
                return carry
            lax.fori_loop(0, NCHUNK, body, 0)
    plsc.subcore_barrier()
    for cc, outh in ((0, cnt0_h), (1, cnt1_h)):
        @pl.when(c == cc)
        def _(outh=outh):
            pltpu.sync_copy(CNT.at[pl.ds(s * RPT, RPT)],
                            outh.at[pl.ds(s * RPT, RPT)])


NC2 = NCHUNK // 2


@functools.cache
def _sc_scatter_kernel():
    return pl.kernel(
        _sc_scatter_body,
        out_type=(jax.ShapeDtypeStruct((NP, H), jnp.float32),
                  jax.ShapeDtypeStruct((NP, H), jnp.float32)),
        mesh=_mesh(),
        scratch_types=[
            pltpu.VMEM((NC2, C), jnp.int32),
            pltpu.VMEM((NC2, C), jnp.int32),
            pltpu.VMEM((C, H), jnp.float32),
            pltpu.VMEM((C, H), jnp.float32),
            pltpu.SemaphoreType.DMA,
            pltpu.SemaphoreType.DMA,
            pltpu.SemaphoreType.DMA,
            pltpu.SemaphoreType.DMA,
            pltpu.VMEM_SHARED((NP, H), jnp.float32),
        ],
    )


def _sc_scatter(row, col, zpa, zpb, zerosH):
    return _sc_scatter_kernel()(row, col, zpa, zpb, zerosH)


def _sc_scatter_body(row_h, col_h, zpa_h, zpb_h, zeros_h, ua_h, ub_h,
                     rowv, colv, buf0, buf1, sg0, sg1, ss0, ss1, ACC):
    c = lax.axis_index("c")
    s = lax.axis_index("s")
    pltpu.sync_copy(zeros_h, ACC.at[pl.ds(s * RPT, RPT)])
    plsc.subcore_barrier()
    bufs = (buf0, buf1)
    gsems = (sg0, sg1)
    ssems = (ss0, ss1)
    for cc, zph in ((0, zpa_h), (1, zpb_h)):
        @pl.when(c == cc)
        def _(zph=zph):
            def start_g(chunk, b):
                pltpu.async_copy(zph.at[rowv.at[chunk]], bufs[b], gsems[b])

            def wait_g(b):
                pltpu.make_async_copy(zph.at[pl.ds(0, C)], bufs[b],
                                      gsems[b]).wait()

            def start_s(chunk, b):
                pltpu.async_copy(bufs[b], ACC.at[colv.at[chunk]], ssems[b],
                                 add=True)

            def wait_s(b):
                pltpu.make_async_copy(bufs[b], ACC.at[pl.ds(0, C)],
                                      ssems[b]).wait()

            for half in range(2):
                pltpu.sync_copy(row_h.at[s].at[pl.ds(half * NC2, NC2)], rowv)
                pltpu.sync_copy(col_h.at[s].at[pl.ds(half * NC2, NC2)], colv)
                start_g(0, 0)
                start_g(1, 1)

                def body(kk, carry):
                    k0 = kk * 2
                    for b in range(2):
                        chunk = k0 + b
                        wait_g(b)
                        start_s(chunk, b)

                        @pl.when(chunk + 2 < NC2)
                        def _s(chunk=chunk, b=b):
                            wait_s(b)
                            start_g(chunk + 2, b)
                    return carry
                lax.fori_loop(0, NC2 // 2, body, 0)
                wait_s(0)
                wait_s(1)
    plsc.subcore_barrier()
    for cc, outh in ((0, ua_h), (1, ub_h)):
        @pl.when(c == cc)
        def _(outh=outh):
            pltpu.sync_copy(ACC.at[pl.ds(s * RPT, RPT)],
                            outh.at[pl.ds(s * RPT, RPT)])


def _dis(cnt_blk):
    return lax.rsqrt(1.0 + cnt_blk[:, :1])


def _mm_scale_body(x_ref, w_ref, cnt_ref, oa_ref, ob_ref):
    xw = jnp.dot(x_ref[...], w_ref[...], preferred_element_type=jnp.float32)
    zp = _dis(cnt_ref[...]) * xw
    oa_ref[...] = zp[:, :H]
    ob_ref[...] = zp[:, H:]


def _mm_scale(x, W, cnt):
    return pl.pallas_call(
        _mm_scale_body,
        grid=(N // BM,),
        in_specs=[
            pl.BlockSpec((BM, F), lambda i: (i, 0)),
            pl.BlockSpec((F, F), lambda i: (0, 0)),
            pl.BlockSpec((BM, H), lambda i: (i, 0)),
        ],
        out_specs=[pl.BlockSpec((BM, H), lambda i: (i, 0))] * 2,
        out_shape=[jax.ShapeDtypeStruct((N, H), jnp.float32)] * 2,
    )(x, W, cnt)


def _mid_body(u0a, u0b, z0a, z0b, u1a, u1b, z1a, z1b, cnt0, cnt1,
              b10, b11, w2, oa_ref, ob_ref):
    dis0 = _dis(cnt0[...])
    dis1 = _dis(cnt1[...])
    h0 = jnp.concatenate([u0a[...] + z0a[...], u0b[...] + z0b[...]], axis=1)
    h0 = jnp.maximum(dis0 * h0 + b10[...], 0.0)
    h1 = jnp.concatenate([u1a[...] + z1a[...], u1b[...] + z1b[...]], axis=1)
    h1 = jnp.maximum(dis1 * h1 + b11[...], 0.0)
    w2 = w2[...]
    t = (jnp.dot(h0, w2[:F], preferred_element_type=jnp.float32)
         + jnp.dot(h1, w2[F:], preferred_element_type=jnp.float32))
    zp2 = dis1 * t
    oa_ref[...] = zp2[:, :H]
    ob_ref[...] = zp2[:, H:]


def _mid(u0a, u0b, z0a, z0b, u1a, u1b, z1a, z1b, cnt0, cnt1, b10, b11, W2):
    blkH = pl.BlockSpec((BM, H), lambda i: (i, 0))
    blkC = pl.BlockSpec((BM, H), lambda i: (i, 0))
    return pl.pallas_call(
        _mid_body,
        grid=(N // BM,),
        in_specs=[blkH] * 8 + [blkC, blkC,
                               pl.BlockSpec((1, F), lambda i: (0, 0)),
                               pl.BlockSpec((1, F), lambda i: (0, 0)),
                               pl.BlockSpec((2 * F, F), lambda i: (0, 0))],
        out_specs=[blkH] * 2,
        out_shape=[jax.ShapeDtypeStruct((N, H), jnp.float32)] * 2,
    )(u0a, u0b, z0a, z0b, u1a, u1b, z1a, z1b, cnt0, cnt1, b10, b11, W2)


def _final_body(u2a, u2b, z2a, z2b, cnt1, b2, o_ref):
    t = jnp.concatenate([u2a[...] + z2a[...], u2b[...] + z2b[...]], axis=1)
    o_ref[...] = _dis(cnt1[...]) * t + b2[...]


def _final(u2a, u2b, z2a, z2b, cnt1, b2):
    blkH = pl.BlockSpec((BM, H), lambda i: (i, 0))
    return pl.pallas_call(
        _final_body,
        grid=(N // BM,),
        in_specs=[blkH] * 4 + [pl.BlockSpec((BM, H), lambda i: (i, 0)),
                               pl.BlockSpec((1, F), lambda i: (0, 0))],
        out_specs=pl.BlockSpec((BM, F), lambda i: (i, 0)),
        out_shape=jax.ShapeDtypeStruct((N, F), jnp.float32),
    )(u2a, u2b, z2a, z2b, cnt1, b2)


def _pad_edges(ei):
    pr = jnp.zeros((EPAD - E,), jnp.int32)
    pc = jnp.full((EPAD - E,), N, jnp.int32)
    row = jnp.concatenate([ei[0], pr]).reshape(NS, NCHUNK, C)
    col = jnp.concatenate([ei[1], pc]).reshape(NS, NCHUNK, C)
    return row, col


def kernel(x0, x1, edge_index0, edge_index1, W1_0, b1_0, W1_1, b1_1, W2, b2):
    row0, col0 = _pad_edges(edge_index0)
    row1, col1 = _pad_edges(edge_index1)
    onesH = jnp.ones((C, H), jnp.float32)
    zerosH = jnp.zeros((RPT, H), jnp.float32)
    b10 = b1_0.reshape(1, F)
    b11 = b1_1.reshape(1, F)
    b2r = b2.reshape(1, F)

    cnt0, cnt1 = _sc_hist(col0, col1, onesH, zerosH)
    cnt0, cnt1 = cnt0[:N], cnt1[:N]
    z0a, z0b = _mm_scale(x0, W1_0, cnt0)
    z1a, z1b = _mm_scale(x1, W1_1, cnt1)
    u0a, u0b = _sc_scatter(row0, col0, z0a, z0b, zerosH)
    u1a, u1b = _sc_scatter(row1, col1, z1a, z1b, zerosH)
    z2a, z2b = _mid(u0a[:N], u0b[:N], z0a, z0b, u1a[:N], u1b[:N], z1a, z1b,
                    cnt0, cnt1, b10, b11, W2)
    u2a, u2b = _sc_scatter(row1, col1, z2a, z2b, zerosH)
    return _final(u2a[:N], u2b[:N], z2a, z2b, cnt1, b2r)
